# baseline (speedup 1.0000x reference)
import functools

import jax
import jax.numpy as jnp
from jax import lax
from jax.experimental import pallas as pl
from jax.experimental.pallas import tpu as pltpu
from jax.experimental.pallas import tpu_sc as plsc

N = 1000000
C = 100
B = 16384
D = 512

NTILES = N // 8

NC = 2
NS = 16
NW = NC * NS
BPW = B // NW

_sc_mesh = plsc.VectorSubcoreMesh(core_axis_name="c", subcore_axis_name="s")


L = 16


def _lane(vec, l):
    return jnp.max(jnp.where(lax.iota(jnp.int32, L) == l, vec, -1))


@functools.partial(
    pl.kernel,
    mesh=_sc_mesh,
    out_type=jax.ShapeDtypeStruct((B, C), jnp.float32),
    compiler_params=pltpu.CompilerParams(needs_layout_passes=False),
    scratch_types=[
        pltpu.VMEM((BPW,), jnp.int32),
        pltpu.VMEM((BPW, C), jnp.float32),
        pltpu.SemaphoreType.DMA,
    ],
)
def _sc_gather(conf3_hbm, idx_hbm, out_hbm, idx_v, rows_v, sem):
    wid = lax.axis_index("s") * NC + lax.axis_index("c")
    base = wid * BPW
    pltpu.sync_copy(idx_hbm.at[pl.ds(base, BPW)], idx_v)

    def issue(k, carry):
        vec = idx_v[pl.ds(k * L, L)]
        for l in range(L):
            v = _lane(vec, l)
            pltpu.make_async_copy(
                conf3_hbm.at[v // 8, v % 8], rows_v.at[k * L + l], sem
            ).start()
        return carry

    lax.fori_loop(0, BPW // L, issue, 0)

    def drain(u, carry):
        pltpu.make_async_copy(conf3_hbm.at[0, 0], rows_v.at[0], sem).wait()
        return carry

    lax.fori_loop(0, BPW, drain, 0)
    pltpu.sync_copy(rows_v, out_hbm.at[pl.ds(base, BPW)])


_FB = 16
_RB = B // _FB


def _bias_body(feat_ref, w_ref, b_ref, blog_ref, acc_ref):
    i = pl.program_id(0)

    @pl.when(i == 0)
    def _():
        acc_ref[...] = jnp.zeros_like(acc_ref)

    acc_ref[...] += jnp.sum(feat_ref[...], axis=0, keepdims=True)

    @pl.when(i == pl.num_programs(0) - 1)
    def _():
        fm = acc_ref[...] * (0.1 / B)
        z = jnp.dot(fm, w_ref[...], preferred_element_type=jnp.float32)
        z = z + b_ref[...]
        z = z - jnp.max(z, axis=-1, keepdims=True)
        e = jnp.exp(z)
        sm = e / jnp.sum(e, axis=-1, keepdims=True)
        blog_ref[...] = jnp.log(sm + 1e-9)


def _rows_body(o1_ref, tgt_ref, blog_ref, nt_ref, loss_ref, acc_ref):
    i = pl.program_id(0)
    x = o1_ref[...]
    t = tgt_ref[...]
    m = jnp.max(x, axis=-1, keepdims=True)
    e = jnp.exp(x - m)
    s = jnp.sum(e, axis=-1, keepdims=True)
    logsm = (x - m) - jnp.log(s)

    @pl.when(i == 0)
    def _():
        acc_ref[0] = 0.0

    acc_ref[0] += jnp.sum(t * logsm)

    x2 = x - blog_ref[...]
    m2 = jnp.max(x2, axis=-1, keepdims=True)
    e2 = jnp.exp(x2 - m2)
    r2 = jnp.where(t > 0, e2, 0.0)
    nt_ref[...] = r2 / jnp.sum(r2, axis=-1, keepdims=True)

    @pl.when(i == pl.num_programs(0) - 1)
    def _():
        loss_ref[0, 0] = -acc_ref[0] / B


_MR = 20000
_MB = N // _MR


def _merge_body(sidx_ref, sorder_ref, starts_ref, conf_ref, nt_ref, out_ref):
    i = pl.program_id(0)
    out_ref[...] = conf_ref[...]
    base = i * _MR

    def body(u, carry):
        r = sidx_ref[u] - base
        b = sorder_ref[u]
        out_ref[pl.ds(r, 1), :] = nt_ref[pl.ds(b, 1), :]
        return carry

    lax.fori_loop(starts_ref[i], starts_ref[i + 1], body, 0)


def kernel(output1, feat, confidence, fc_W, fc_b, index):
    idx32 = index.astype(jnp.int32)
    conf3 = confidence.reshape(NTILES, 8, C)

    target = _sc_gather(conf3, idx32)

    blog = pl.pallas_call(
        _bias_body,
        grid=(_FB,),
        in_specs=[
            pl.BlockSpec((_RB, D), lambda i: (i, 0)),
            pl.BlockSpec((D, C), lambda i: (0, 0)),
            pl.BlockSpec((1, C), lambda i: (0, 0)),
        ],
        out_specs=pl.BlockSpec((1, C), lambda i: (0, 0)),
        out_shape=jax.ShapeDtypeStruct((1, C), jnp.float32),
        scratch_shapes=[pltpu.VMEM((1, D), jnp.float32)],
    )(feat, fc_W, fc_b.reshape(1, C))

    new_target, loss = pl.pallas_call(
        _rows_body,
        grid=(_FB,),
        in_specs=[
            pl.BlockSpec((_RB, C), lambda i: (i, 0)),
            pl.BlockSpec((_RB, C), lambda i: (i, 0)),
            pl.BlockSpec((1, C), lambda i: (0, 0)),
        ],
        out_specs=[
            pl.BlockSpec((_RB, C), lambda i: (i, 0)),
            pl.BlockSpec(memory_space=pltpu.SMEM),
        ],
        out_shape=[
            jax.ShapeDtypeStruct((B, C), jnp.float32),
            jax.ShapeDtypeStruct((1, 1), jnp.float32),
        ],
        scratch_shapes=[pltpu.SMEM((1,), jnp.float32)],
    )(output1, target, blog)

    sorder = jnp.argsort(idx32, stable=True).astype(jnp.int32)
    sidx = idx32[sorder]
    starts = jnp.searchsorted(
        sidx, jnp.arange(_MB + 1, dtype=jnp.int32) * _MR
    ).astype(jnp.int32)

    new_confidence = pl.pallas_call(
        _merge_body,
        grid_spec=pltpu.PrefetchScalarGridSpec(
            num_scalar_prefetch=3,
            grid=(_MB,),
            in_specs=[
                pl.BlockSpec((_MR, C), lambda i, *_: (i, 0)),
                pl.BlockSpec((B, C), lambda i, *_: (0, 0)),
            ],
            out_specs=pl.BlockSpec((_MR, C), lambda i, *_: (i, 0)),
        ),
        out_shape=jax.ShapeDtypeStruct((N, C), jnp.float32),
    )(sidx, sorder, starts, confidence, new_target)

    return loss[0, 0], new_confidence


# speedup vs baseline: 1.0222x; 1.0222x over previous
import functools

import jax
import jax.numpy as jnp
from jax import lax
from jax.experimental import pallas as pl
from jax.experimental.pallas import tpu as pltpu
from jax.experimental.pallas import tpu_sc as plsc

N = 1000000
C = 100
B = 16384
D = 512

NTILES = N // 8

NC = 2
NS = 16
NW = NC * NS
BPW = B // NW

_sc_mesh = plsc.VectorSubcoreMesh(core_axis_name="c", subcore_axis_name="s")


L = 16


def _lane(vec, l):
    return jnp.max(jnp.where(lax.iota(jnp.int32, L) == l, vec, -1))


@functools.partial(
    pl.kernel,
    mesh=_sc_mesh,
    out_type=jax.ShapeDtypeStruct((B, C), jnp.float32),
    compiler_params=pltpu.CompilerParams(needs_layout_passes=False),
    scratch_types=[
        pltpu.VMEM((BPW,), jnp.int32),
        pltpu.VMEM((BPW, C), jnp.float32),
        pltpu.SemaphoreType.DMA,
    ],
)
def _sc_gather(conf3_hbm, idx_hbm, out_hbm, idx_v, rows_v, sem):
    wid = lax.axis_index("s") * NC + lax.axis_index("c")
    base = wid * BPW
    pltpu.sync_copy(idx_hbm.at[pl.ds(base, BPW)], idx_v)

    def issue(k, carry):
        vec = idx_v[pl.ds(k * L, L)]
        for l in range(L):
            v = _lane(vec, l)
            pltpu.make_async_copy(
                conf3_hbm.at[v // 8, v % 8], rows_v.at[k * L + l], sem
            ).start()
        return carry

    lax.fori_loop(0, BPW // L, issue, 0)

    def drain(u, carry):
        pltpu.make_async_copy(conf3_hbm.at[0, 0], rows_v.at[0], sem).wait()
        return carry

    lax.fori_loop(0, BPW, drain, 0)
    pltpu.sync_copy(rows_v, out_hbm.at[pl.ds(base, BPW)])


_FB = 8
_RB = B // _FB


def _bias_body(feat_ref, w_ref, b_ref, blog_ref, acc_ref):
    i = pl.program_id(0)

    @pl.when(i == 0)
    def _():
        acc_ref[...] = jnp.zeros_like(acc_ref)

    acc_ref[...] += jnp.sum(feat_ref[...], axis=0, keepdims=True)

    @pl.when(i == pl.num_programs(0) - 1)
    def _():
        fm = acc_ref[...] * (0.1 / B)
        z = jnp.dot(fm, w_ref[...], preferred_element_type=jnp.float32)
        z = z + b_ref[...]
        z = z - jnp.max(z, axis=-1, keepdims=True)
        e = jnp.exp(z)
        sm = e / jnp.sum(e, axis=-1, keepdims=True)
        blog_ref[...] = jnp.log(sm + 1e-9)


def _rows_body(o1_ref, tgt_ref, blog_ref, nt_ref, loss_ref, acc_ref):
    i = pl.program_id(0)
    x = o1_ref[...]
    t = tgt_ref[...]
    m = jnp.max(x, axis=-1, keepdims=True)
    e = jnp.exp(x - m)
    s = jnp.sum(e, axis=-1, keepdims=True)
    logsm = (x - m) - jnp.log(s)

    @pl.when(i == 0)
    def _():
        acc_ref[0] = 0.0

    acc_ref[0] += jnp.sum(t * logsm)

    x2 = x - blog_ref[...]
    m2 = jnp.max(x2, axis=-1, keepdims=True)
    e2 = jnp.exp(x2 - m2)
    r2 = jnp.where(t > 0, e2, 0.0)
    nt_ref[...] = r2 / jnp.sum(r2, axis=-1, keepdims=True)

    @pl.when(i == pl.num_programs(0) - 1)
    def _():
        loss_ref[0, 0] = -acc_ref[0] / B


_MR = 20000
_MB = N // _MR


def _merge_body(sidx_ref, sorder_ref, starts_ref, conf_ref, nt_ref, out_ref):
    i = pl.program_id(0)
    out_ref[...] = conf_ref[...]
    base = i * _MR

    def body(u, carry):
        r = sidx_ref[u] - base
        b = sorder_ref[u]
        out_ref[pl.ds(r, 1), :] = nt_ref[pl.ds(b, 1), :]
        return carry

    lax.fori_loop(starts_ref[i], starts_ref[i + 1], body, 0)


def kernel(output1, feat, confidence, fc_W, fc_b, index):
    idx32 = index.astype(jnp.int32)
    conf3 = confidence.reshape(NTILES, 8, C)

    target = _sc_gather(conf3, idx32)

    blog = pl.pallas_call(
        _bias_body,
        grid=(_FB,),
        in_specs=[
            pl.BlockSpec((_RB, D), lambda i: (i, 0)),
            pl.BlockSpec((D, C), lambda i: (0, 0)),
            pl.BlockSpec((1, C), lambda i: (0, 0)),
        ],
        out_specs=pl.BlockSpec((1, C), lambda i: (0, 0)),
        out_shape=jax.ShapeDtypeStruct((1, C), jnp.float32),
        scratch_shapes=[pltpu.VMEM((1, D), jnp.float32)],
    )(feat, fc_W, fc_b.reshape(1, C))

    new_target, loss = pl.pallas_call(
        _rows_body,
        grid=(_FB,),
        in_specs=[
            pl.BlockSpec((_RB, C), lambda i: (i, 0)),
            pl.BlockSpec((_RB, C), lambda i: (i, 0)),
            pl.BlockSpec((1, C), lambda i: (0, 0)),
        ],
        out_specs=[
            pl.BlockSpec((_RB, C), lambda i: (i, 0)),
            pl.BlockSpec(memory_space=pltpu.SMEM),
        ],
        out_shape=[
            jax.ShapeDtypeStruct((B, C), jnp.float32),
            jax.ShapeDtypeStruct((1, 1), jnp.float32),
        ],
        scratch_shapes=[pltpu.SMEM((1,), jnp.float32)],
    )(output1, target, blog)

    sidx, sorder = lax.sort(
        (idx32, jnp.arange(B, dtype=jnp.int32)), num_keys=1, is_stable=True
    )
    starts = jnp.searchsorted(
        sidx, jnp.arange(_MB + 1, dtype=jnp.int32) * _MR
    ).astype(jnp.int32)

    new_confidence = pl.pallas_call(
        _merge_body,
        grid_spec=pltpu.PrefetchScalarGridSpec(
            num_scalar_prefetch=3,
            grid=(_MB,),
            in_specs=[
                pl.BlockSpec((_MR, C), lambda i, *_: (i, 0)),
                pl.BlockSpec((B, C), lambda i, *_: (0, 0)),
            ],
            out_specs=pl.BlockSpec((_MR, C), lambda i, *_: (i, 0)),
        ),
        out_shape=jax.ShapeDtypeStruct((N, C), jnp.float32),
    )(sidx, sorder, starts, confidence, new_target)

    return loss[0, 0], new_confidence
